# baseline (speedup 1.0000x reference)
import functools

import jax
import jax.numpy as jnp
from jax import lax
from jax.experimental import pallas as pl
from jax.experimental.pallas import tpu as pltpu
from jax.experimental.pallas import tpu_sc as plsc

VOCAB = 1000000
EMB = 32
SEQ = 50

_INFO = plsc.get_sparse_core_info()
NC = _INFO.num_cores
NS = _INFO.num_subcores
NW = NC * NS

CHUNK_B = 1024
TC_PER_CHUNK = CHUNK_B // 128


def _gather_body(n_chunks_pw, tj_groups, b_total, idx_hbm, table_hbm, o7_hbm,
                 idx_v, buf_v, trows_v, gsem, osem):
  c = lax.axis_index("c")
  s = lax.axis_index("s")
  wid = s * NC + c

  def chunk_body(i, carry):
    k = wid * n_chunks_pw + i
    s_id = k // tj_groups
    tj0 = (k % tj_groups) * TC_PER_CHUNK
    off = s_id * b_total + tj0 * 128
    pltpu.sync_copy(idx_hbm.at[pl.ds(off, CHUNK_B)], idx_v)
    pltpu.async_copy(table_hbm.at[idx_v], buf_v, gsem).wait()

    def tc_body(tc, carry2):
      row0 = tc * 128
      lane_e = lax.iota(jnp.int32, 16) * 128
      for rr in range(128):
        r = row0 + rr
        lo = buf_v[r, pl.ds(0, 16)]
        hi = buf_v[r, pl.ds(16, 16)]
        plsc.store_scatter(trows_v, [lane_e + rr], lo)
        plsc.store_scatter(trows_v, [lane_e + (16 * 128 + rr)], hi)
      pend = [
          pltpu.async_copy(trows_v.at[pl.ds(ti * 1024, 1024)],
                           o7_hbm.at[s_id, ti, tj0 + tc], osem)
          for ti in range(4)
      ]
      for p in pend:
        p.wait()
      return carry2

    lax.fori_loop(0, TC_PER_CHUNK, tc_body, 0)
    return carry

  lax.fori_loop(0, n_chunks_pw, chunk_body, 0)


def kernel(inputs, embeddings):
  b, seq = inputs.shape
  assert seq == SEQ and b % 128 == 0
  tj_all = b // 128
  assert tj_all % TC_PER_CHUNK == 0
  tj_groups = tj_all // TC_PER_CHUNK
  n_chunks = SEQ * tj_groups
  assert n_chunks % NW == 0
  n_chunks_pw = n_chunks // NW

  idx_sm = jnp.reshape(inputs.T, (-1,)).astype(jnp.int32)

  mesh = plsc.VectorSubcoreMesh(core_axis_name="c", subcore_axis_name="s")
  gather = pl.kernel(
      functools.partial(_gather_body, n_chunks_pw, tj_groups, b),
      out_type=jax.ShapeDtypeStruct((SEQ, 4, tj_all, 1024), jnp.float32),
      mesh=mesh,
      scratch_types=[
          pltpu.VMEM((CHUNK_B,), jnp.int32),
          pltpu.VMEM((CHUNK_B, EMB), jnp.float32),
          pltpu.VMEM((EMB * 128,), jnp.float32),
          pltpu.SemaphoreType.DMA,
          pltpu.SemaphoreType.DMA,
      ],
      compiler_params=pltpu.CompilerParams(use_tc_tiling_on_sc=False,
                                           needs_layout_passes=False),
  )
  o7 = gather(idx_sm, embeddings)
  o7b = jnp.reshape(o7, (SEQ, 4, tj_all, 8, 128))
  o5 = jnp.transpose(o7b, (0, 1, 3, 2, 4))
  o3 = jnp.reshape(o5, (SEQ, EMB, b))
  return jnp.transpose(o3, (2, 0, 1))


# speedup vs baseline: 1.0249x; 1.0249x over previous
import functools

import jax
import jax.numpy as jnp
from jax import lax
from jax.experimental import pallas as pl
from jax.experimental.pallas import tpu as pltpu
from jax.experimental.pallas import tpu_sc as plsc

VOCAB = 1000000
EMB = 32
SEQ = 50

_INFO = plsc.get_sparse_core_info()
NC = _INFO.num_cores
NS = _INFO.num_subcores
NW = NC * NS

CHUNK_B = 1024
TC_PER_CHUNK = CHUNK_B // 128


def _gather_body(n_chunks_pw, tj_groups, b_total, idx_hbm, table_hbm, o7_hbm,
                 idx_v, buf_v, trows_v0, trows_v1, gsem, osem):
  c = lax.axis_index("c")
  s = lax.axis_index("s")
  wid = s * NC + c
  trows_bufs = [trows_v0, trows_v1]

  def chunk_body(i, carry):
    k = wid * n_chunks_pw + i
    s_id = k // tj_groups
    tj0 = (k % tj_groups) * TC_PER_CHUNK
    off = s_id * b_total + tj0 * 128
    pltpu.sync_copy(idx_hbm.at[pl.ds(off, CHUNK_B)], idx_v)
    pltpu.async_copy(table_hbm.at[idx_v], buf_v, gsem).wait()

    lane_e = lax.iota(jnp.int32, 16) * 128
    pend = [None, None]
    for tc in range(TC_PER_CHUNK):
      tb = trows_bufs[tc % 2]
      if pend[tc % 2] is not None:
        for p in pend[tc % 2]:
          p.wait()
      for rr in range(128):
        r = tc * 128 + rr
        lo = buf_v[r, pl.ds(0, 16)]
        hi = buf_v[r, pl.ds(16, 16)]
        plsc.store_scatter(tb, [lane_e + rr], lo)
        plsc.store_scatter(tb, [lane_e + (16 * 128 + rr)], hi)
      pend[tc % 2] = [
          pltpu.async_copy(tb.at[pl.ds(ti * 1024, 1024)],
                           o7_hbm.at[s_id, ti, tj0 + tc], osem)
          for ti in range(4)
      ]
    for plist in pend:
      if plist is not None:
        for p in plist:
          p.wait()
    return carry

  lax.fori_loop(0, n_chunks_pw, chunk_body, 0)


def kernel(inputs, embeddings):
  b, seq = inputs.shape
  assert seq == SEQ and b % 128 == 0
  tj_all = b // 128
  assert tj_all % TC_PER_CHUNK == 0
  tj_groups = tj_all // TC_PER_CHUNK
  n_chunks = SEQ * tj_groups
  assert n_chunks % NW == 0
  n_chunks_pw = n_chunks // NW

  idx_sm = jnp.reshape(inputs.T, (-1,)).astype(jnp.int32)

  mesh = plsc.VectorSubcoreMesh(core_axis_name="c", subcore_axis_name="s")
  gather = pl.kernel(
      functools.partial(_gather_body, n_chunks_pw, tj_groups, b),
      out_type=jax.ShapeDtypeStruct((SEQ, 4, tj_all, 1024), jnp.float32),
      mesh=mesh,
      scratch_types=[
          pltpu.VMEM((CHUNK_B,), jnp.int32),
          pltpu.VMEM((CHUNK_B, EMB), jnp.float32),
          pltpu.VMEM((EMB * 128,), jnp.float32),
          pltpu.VMEM((EMB * 128,), jnp.float32),
          pltpu.SemaphoreType.DMA,
          pltpu.SemaphoreType.DMA,
      ],
      compiler_params=pltpu.CompilerParams(use_tc_tiling_on_sc=False,
                                           needs_layout_passes=False),
  )
  o7 = gather(idx_sm, embeddings)
  o7b = jnp.reshape(o7, (SEQ, 4, tj_all, 8, 128))
  o5 = jnp.transpose(o7b, (0, 1, 3, 2, 4))
  o3 = jnp.reshape(o5, (SEQ, EMB, b))
  return jnp.transpose(o3, (2, 0, 1))
